# baseline (speedup 1.0000x reference)
import jax
import jax.numpy as jnp
from jax import lax
from jax.experimental import pallas as pl
from jax.experimental.pallas import tpu as pltpu
from jax.experimental.pallas import tpu_sc as plsc

N = 10000
D = 128
NC = 2
NS = 16
C = 128
N_PAD = 10112
ROWS_PER_TILE = N_PAD // NS
SUB = [(k * C, min(C, ROWS_PER_TILE - k * C))
       for k in range((ROWS_PER_TILE + C - 1) // C)]
BM = 2000


def _mesh():
    return plsc.VectorSubcoreMesh(core_axis_name="c", subcore_axis_name="s")


def _zero_acc(zf_hbm, rows_v, acc, r0):
    pltpu.sync_copy(zf_hbm.at[pl.ds(0, C)], rows_v)
    for o, w in SUB:
        pltpu.sync_copy(rows_v.at[pl.ds(0, w)], acc.at[pl.ds(r0 + o, w)])


def _write_acc(acc, rows_v, part_out, r0, obase):
    for o, w in SUB:
        pltpu.sync_copy(acc.at[pl.ds(r0 + o, w)], rows_v.at[pl.ds(0, w)])
        pltpu.sync_copy(rows_v.at[pl.ds(0, w)],
                        part_out.at[pl.ds(obase + o, w)])


def _make_sc_spmm(cpt0: int, cpt1: int):
    def body(table, src_hbm, dst_hbm, zf_hbm, part_out,
             acc, src_v, dst_v, rows_v, gsem):
        cid = lax.axis_index("c")
        sid = lax.axis_index("s")
        r0 = sid * ROWS_PER_TILE
        _zero_acc(zf_hbm, rows_v, acc, r0)
        plsc.subcore_barrier()

        my_cpt = jnp.where(cid == 0, cpt0, cpt1)
        ebase = jnp.where(cid == 0, sid * cpt0,
                          NS * cpt0 + sid * cpt1) * C

        def chunk(j, carry):
            off = ebase + j * C
            pltpu.sync_copy(src_hbm.at[pl.ds(off, C)], src_v)
            pltpu.sync_copy(dst_hbm.at[pl.ds(off, C)], dst_v)
            pltpu.async_copy(table.at[src_v], rows_v, gsem).wait()
            pltpu.sync_copy(rows_v, acc.at[dst_v], add=True)
            return carry

        lax.fori_loop(0, my_cpt, chunk, 0)
        plsc.subcore_barrier()
        _write_acc(acc, rows_v, part_out, r0, cid * N_PAD + r0)

    return pl.kernel(
        body,
        out_type=jax.ShapeDtypeStruct((NC * N_PAD, D), jnp.float32),
        mesh=_mesh(),
        scratch_types=[
            pltpu.VMEM_SHARED((N_PAD, D), jnp.float32),
            pltpu.VMEM((C,), jnp.int32),
            pltpu.VMEM((C,), jnp.int32),
            pltpu.VMEM((C, D), jnp.float32),
            pltpu.SemaphoreType.DMA,
        ],
    )


def _make_sc_count(chunks_per_tile: int):
    def body(dst_hbm, zf_hbm, ones_hbm, cnt_out,
             acc, dst_v, ones_v, rows_v):
        cid = lax.axis_index("c")
        sid = lax.axis_index("s")
        r0 = sid * ROWS_PER_TILE
        _zero_acc(zf_hbm, rows_v, acc, r0)
        pltpu.sync_copy(ones_hbm, ones_v)
        plsc.subcore_barrier()

        ebase = (cid * NS + sid) * chunks_per_tile * C

        def chunk(j, carry):
            off = ebase + j * C
            pltpu.sync_copy(dst_hbm.at[pl.ds(off, C)], dst_v)
            pltpu.sync_copy(ones_v, acc.at[dst_v], add=True)
            return carry

        lax.fori_loop(0, chunks_per_tile, chunk, 0)
        plsc.subcore_barrier()
        _write_acc(acc, rows_v, cnt_out, r0, cid * N_PAD + r0)

    return pl.kernel(
        body,
        out_type=jax.ShapeDtypeStruct((NC * N_PAD, D), jnp.float32),
        mesh=_mesh(),
        scratch_types=[
            pltpu.VMEM_SHARED((N_PAD, D), jnp.float32),
            pltpu.VMEM((C,), jnp.int32),
            pltpu.VMEM((C, D), jnp.float32),
            pltpu.VMEM((C, D), jnp.float32),
        ],
    )


def _combine_mm(aggp, cntp, feat, w_cat, b_row, relu: bool):
    def body(agg_ref, cnt_ref, feat_ref, w_ref, b_ref, o_ref):
        agg = agg_ref[0] + agg_ref[1]
        cnt = cnt_ref[0, :, 0:1] + cnt_ref[1, :, 0:1]
        mean = agg / jnp.maximum(cnt, 1.0)
        cat = jnp.concatenate([mean, feat_ref[...]], axis=1)
        out = jnp.dot(cat, w_ref[...], preferred_element_type=jnp.float32)
        out = out + b_ref[...]
        o_ref[...] = jnp.maximum(out, 0.0) if relu else out

    return pl.pallas_call(
        body,
        grid=(N // BM,),
        in_specs=[
            pl.BlockSpec((NC, BM, D), lambda i: (0, i, 0)),
            pl.BlockSpec((NC, BM, D), lambda i: (0, i, 0)),
            pl.BlockSpec((BM, D), lambda i: (i, 0)),
            pl.BlockSpec((2 * D, D), lambda i: (0, 0)),
            pl.BlockSpec((1, D), lambda i: (0, 0)),
        ],
        out_specs=pl.BlockSpec((BM, D), lambda i: (i, 0)),
        out_shape=jax.ShapeDtypeStruct((N, D), jnp.float32),
    )(aggp, cntp, feat, w_cat, b_row)


def kernel(x, edge_index, W1_l, b1, W1_r, W2_l, b2, W2_r):
    e = edge_index.shape[1]
    grain = NC * NS * C
    chunks_per_tile = (e + grain - 1) // grain
    e_pad = chunks_per_tile * grain
    cpt0 = int(round(chunks_per_tile * 2 * 0.574))
    cpt1 = 2 * chunks_per_tile - cpt0

    src = edge_index[0].astype(jnp.int32)
    dst = edge_index[1].astype(jnp.int32)
    pad = e_pad - e
    if pad:
        src = jnp.concatenate([src, jnp.zeros((pad,), jnp.int32)])
        dst = jnp.concatenate([dst, jnp.full((pad,), N_PAD - 1, jnp.int32)])

    zf = jnp.zeros((N_PAD, D), jnp.float32)
    ones = jnp.ones((C, D), jnp.float32)

    spmm = _make_sc_spmm(cpt0, cpt1)
    count = _make_sc_count(chunks_per_tile)

    cntp = count(dst, zf, ones).reshape(NC, N_PAD, D)
    agg1p = spmm(x, src, dst, zf).reshape(NC, N_PAD, D)
    w1 = jnp.concatenate([W1_l, W1_r], axis=0)
    h = _combine_mm(agg1p, cntp, x, w1, b1.reshape(1, D), relu=True)
    agg2p = spmm(h, src, dst, zf).reshape(NC, N_PAD, D)
    w2 = jnp.concatenate([W2_l, W2_r], axis=0)
    return _combine_mm(agg2p, cntp, h, w2, b2.reshape(1, D), relu=False)


# speedup vs baseline: 1.0294x; 1.0294x over previous
import jax
import jax.numpy as jnp
from jax import lax
from jax.experimental import pallas as pl
from jax.experimental.pallas import tpu as pltpu
from jax.experimental.pallas import tpu_sc as plsc

N = 10000
D = 128
NC = 2
NS = 16
C = 128
N_PAD = 10112
ROWS_PER_TILE = N_PAD // NS
SUB = [(k * C, min(C, ROWS_PER_TILE - k * C))
       for k in range((ROWS_PER_TILE + C - 1) // C)]
BM = 2000


def _mesh():
    return plsc.VectorSubcoreMesh(core_axis_name="c", subcore_axis_name="s")


def _zero_acc(zf_hbm, rows_v, acc, r0):
    pltpu.sync_copy(zf_hbm.at[pl.ds(0, C)], rows_v)
    for o, w in SUB:
        pltpu.sync_copy(rows_v.at[pl.ds(0, w)], acc.at[pl.ds(r0 + o, w)])


def _write_acc(acc, rows_v, part_out, r0, obase):
    for o, w in SUB:
        pltpu.sync_copy(acc.at[pl.ds(r0 + o, w)], rows_v.at[pl.ds(0, w)])
        pltpu.sync_copy(rows_v.at[pl.ds(0, w)],
                        part_out.at[pl.ds(obase + o, w)])


def _make_sc_spmm(cpt0: int, cpt1: int):
    def body(table, src_hbm, dst_hbm, zf_hbm, part_out,
             acc, src_v, dst_v, rows_v, gsem):
        cid = lax.axis_index("c")
        sid = lax.axis_index("s")
        r0 = sid * ROWS_PER_TILE
        _zero_acc(zf_hbm, rows_v, acc, r0)
        plsc.subcore_barrier()

        my_cpt = jnp.where(cid == 0, cpt0, cpt1)
        ebase = jnp.where(cid == 0, sid * cpt0,
                          NS * cpt0 + sid * cpt1) * C

        def chunk(j, carry):
            off = ebase + j * C
            pltpu.sync_copy(src_hbm.at[pl.ds(off, C)], src_v)
            pltpu.sync_copy(dst_hbm.at[pl.ds(off, C)], dst_v)
            pltpu.async_copy(table.at[src_v], rows_v, gsem).wait()
            pltpu.sync_copy(rows_v, acc.at[dst_v], add=True)
            return carry

        lax.fori_loop(0, my_cpt, chunk, 0)
        plsc.subcore_barrier()
        _write_acc(acc, rows_v, part_out, r0, cid * N_PAD + r0)

    return pl.kernel(
        body,
        out_type=jax.ShapeDtypeStruct((NC * N_PAD, D), jnp.float32),
        mesh=_mesh(),
        scratch_types=[
            pltpu.VMEM_SHARED((N_PAD, D), jnp.float32),
            pltpu.VMEM((C,), jnp.int32),
            pltpu.VMEM((C,), jnp.int32),
            pltpu.VMEM((C, D), jnp.float32),
            pltpu.SemaphoreType.DMA,
        ],
    )


def _make_sc_count(chunks_per_tile: int):
    def body(dst_hbm, zf_hbm, ones_hbm, cnt_out,
             acc, dst_v, ones_v, rows_v):
        cid = lax.axis_index("c")
        sid = lax.axis_index("s")
        r0 = sid * ROWS_PER_TILE
        _zero_acc(zf_hbm, rows_v, acc, r0)
        pltpu.sync_copy(ones_hbm, ones_v)
        plsc.subcore_barrier()

        ebase = (cid * NS + sid) * chunks_per_tile * C

        def chunk(j, carry):
            off = ebase + j * C
            pltpu.sync_copy(dst_hbm.at[pl.ds(off, C)], dst_v)
            pltpu.sync_copy(ones_v, acc.at[dst_v], add=True)
            return carry

        lax.fori_loop(0, chunks_per_tile, chunk, 0)
        plsc.subcore_barrier()
        _write_acc(acc, rows_v, cnt_out, r0, cid * N_PAD + r0)

    return pl.kernel(
        body,
        out_type=jax.ShapeDtypeStruct((NC * N_PAD, D), jnp.float32),
        mesh=_mesh(),
        scratch_types=[
            pltpu.VMEM_SHARED((N_PAD, D), jnp.float32),
            pltpu.VMEM((C,), jnp.int32),
            pltpu.VMEM((C, D), jnp.float32),
            pltpu.VMEM((C, D), jnp.float32),
        ],
    )


def _combine_mm(aggp, cntp, feat, w_cat, b_row, relu: bool):
    def body(agg_ref, cnt_ref, feat_ref, w_ref, b_ref, o_ref):
        agg = agg_ref[0] + agg_ref[1]
        cnt = cnt_ref[0, :, 0:1] + cnt_ref[1, :, 0:1]
        mean = agg / jnp.maximum(cnt, 1.0)
        cat = jnp.concatenate([mean, feat_ref[...]], axis=1)
        out = jnp.dot(cat, w_ref[...], preferred_element_type=jnp.float32)
        out = out + b_ref[...]
        o_ref[...] = jnp.maximum(out, 0.0) if relu else out

    return pl.pallas_call(
        body,
        grid=(N // BM,),
        in_specs=[
            pl.BlockSpec((NC, BM, D), lambda i: (0, i, 0)),
            pl.BlockSpec((NC, BM, D), lambda i: (0, i, 0)),
            pl.BlockSpec((BM, D), lambda i: (i, 0)),
            pl.BlockSpec((2 * D, D), lambda i: (0, 0)),
            pl.BlockSpec((1, D), lambda i: (0, 0)),
        ],
        out_specs=pl.BlockSpec((BM, D), lambda i: (i, 0)),
        out_shape=jax.ShapeDtypeStruct((N, D), jnp.float32),
    )(aggp, cntp, feat, w_cat, b_row)


def kernel(x, edge_index, W1_l, b1, W1_r, W2_l, b2, W2_r):
    e = edge_index.shape[1]
    grain = NC * NS * C
    chunks_per_tile = (e + grain - 1) // grain
    e_pad = chunks_per_tile * grain
    cpt0 = int(round(chunks_per_tile * 2 * 0.61))
    cpt1 = 2 * chunks_per_tile - cpt0

    src = edge_index[0].astype(jnp.int32)
    dst = edge_index[1].astype(jnp.int32)
    pad = e_pad - e
    if pad:
        src = jnp.concatenate([src, jnp.zeros((pad,), jnp.int32)])
        dst = jnp.concatenate([dst, jnp.full((pad,), N_PAD - 1, jnp.int32)])

    zf = jnp.zeros((N_PAD, D), jnp.float32)
    ones = jnp.ones((C, D), jnp.float32)

    spmm = _make_sc_spmm(cpt0, cpt1)
    count = _make_sc_count(chunks_per_tile)

    cntp = count(dst, zf, ones).reshape(NC, N_PAD, D)
    agg1p = spmm(x, src, dst, zf).reshape(NC, N_PAD, D)
    w1 = jnp.concatenate([W1_l, W1_r], axis=0)
    h = _combine_mm(agg1p, cntp, x, w1, b1.reshape(1, D), relu=True)
    agg2p = spmm(h, src, dst, zf).reshape(NC, N_PAD, D)
    w2 = jnp.concatenate([W2_l, W2_r], axis=0)
    return _combine_mm(agg2p, cntp, h, w2, b2.reshape(1, D), relu=False)
